# baseline (speedup 1.0000x reference)
import functools

import jax
import jax.numpy as jnp
from jax import lax
from jax.experimental import pallas as pl
from jax.experimental.pallas import tpu as pltpu
from jax.experimental.pallas import tpu_sc as plsc

_NUM_CCSK = 8
_SEQ = 128
_F = _NUM_CCSK * _SEQ
_NC = 2
_NS = 16
_NW = _NC * _NS
_CH = 16


def _permute_chunk(in_v, out_v):
    pattern = lax.iota(jnp.int32, 16) << 3

    def q_body(q, _):
        r = q >> 3
        jh = q & 7
        rvec = jnp.broadcast_to(r, (16,))
        vs = [
            in_v[r, pl.ds(k * _SEQ + 16 * jh, 16)]
            for k in range(_NUM_CCSK)
        ]
        for k in range(_NUM_CCSK):
            plsc.store_scatter(
                out_v, [rvec, pattern + (jh * _SEQ + k)], vs[k]
            )
        return 0

    lax.fori_loop(0, _CH * _NUM_CCSK, q_body, 0, unroll=4)


def _sc_body(x_hbm, out_hbm, in_a, in_b, out_a, out_b, si_a, si_b, so_a, so_b):
    wid = lax.axis_index("s") * _NC + lax.axis_index("c")
    rows = x_hbm.shape[0]
    n = rows // _CH // _NW
    r_base = wid * n * _CH

    ins = [in_a, in_b]
    outs = [out_a, out_b]
    sem_in = [si_a, si_b]
    sem_out = [so_a, so_b]

    def start_in(i, b):
        r0 = r_base + i * _CH
        pltpu.async_copy(x_hbm.at[pl.ds(r0, _CH)], ins[b], sem_in[b])

    def start_out(i, b):
        r0 = r_base + i * _CH
        pltpu.async_copy(outs[b], out_hbm.at[pl.ds(r0, _CH)], sem_out[b])

    def wait_in(b):
        pltpu.make_async_copy(
            x_hbm.at[pl.ds(0, _CH)], ins[b], sem_in[b]
        ).wait()

    def wait_out(b):
        pltpu.make_async_copy(
            outs[b], out_hbm.at[pl.ds(0, _CH)], sem_out[b]
        ).wait()

    start_in(0, 0)
    start_in(1, 1)
    wait_in(0)
    _permute_chunk(ins[0], outs[0])
    start_out(0, 0)
    start_in(2, 0)
    wait_in(1)
    _permute_chunk(ins[1], outs[1])
    start_out(1, 1)
    start_in(3, 1)

    def pair_body(g, _):
        i0 = 2 * g
        wait_in(0)
        wait_out(0)
        _permute_chunk(ins[0], outs[0])
        start_out(i0, 0)
        start_in(i0 + 2, 0)
        wait_in(1)
        wait_out(1)
        _permute_chunk(ins[1], outs[1])
        start_out(i0 + 1, 1)
        start_in(i0 + 3, 1)
        return 0

    lax.fori_loop(1, n // 2 - 1, pair_body, 0)

    wait_in(0)
    wait_out(0)
    _permute_chunk(ins[0], outs[0])
    start_out(n - 2, 0)
    wait_in(1)
    wait_out(1)
    _permute_chunk(ins[1], outs[1])
    start_out(n - 1, 1)
    wait_out(0)
    wait_out(1)


def kernel(inputs):
    b, t, f = inputs.shape
    xt = jnp.transpose(inputs, (1, 0, 2)).reshape(t * b, f)
    mesh = plsc.VectorSubcoreMesh(core_axis_name="c", subcore_axis_name="s")
    k = functools.partial(
        pl.kernel,
        out_type=jax.ShapeDtypeStruct((t * b, f), jnp.float32),
        mesh=mesh,
        scratch_types=[
            pltpu.VMEM((_CH, f), jnp.float32),
            pltpu.VMEM((_CH, f), jnp.float32),
            pltpu.VMEM((_CH, f), jnp.float32),
            pltpu.VMEM((_CH, f), jnp.float32),
            pltpu.SemaphoreType.DMA,
            pltpu.SemaphoreType.DMA,
            pltpu.SemaphoreType.DMA,
            pltpu.SemaphoreType.DMA,
        ],
        compiler_params=pltpu.CompilerParams(
            needs_layout_passes=False, use_tc_tiling_on_sc=True
        ),
    )(_sc_body)
    return jnp.transpose(k(xt).reshape(t, b, f), (1, 0, 2))


# speedup vs baseline: 1.0095x; 1.0095x over previous
import functools

import jax
import jax.numpy as jnp
from jax import lax
from jax.experimental import pallas as pl
from jax.experimental.pallas import tpu as pltpu
from jax.experimental.pallas import tpu_sc as plsc

_NUM_CCSK = 8
_SEQ = 128
_F = _NUM_CCSK * _SEQ
_NC = 2
_NS = 16
_NW = _NC * _NS
_CH = 16


def _permute_chunk(in_v, out_v):
    pattern = lax.iota(jnp.int32, 16) << 3

    def q_body(q, _):
        r = q >> 3
        jh = q & 7
        rvec = jnp.broadcast_to(r, (16,))
        vs = [
            in_v[r, pl.ds(k * _SEQ + 16 * jh, 16)]
            for k in range(_NUM_CCSK)
        ]
        for k in range(_NUM_CCSK):
            plsc.store_scatter(
                out_v, [rvec, pattern + (jh * _SEQ + k)], vs[k]
            )
        return 0

    lax.fori_loop(0, _CH * _NUM_CCSK, q_body, 0, unroll=2)


def _sc_body(x_hbm, out_hbm, in_a, in_b, out_a, out_b, si_a, si_b, so_a, so_b):
    wid = lax.axis_index("s") * _NC + lax.axis_index("c")
    rows = x_hbm.shape[0]
    n = rows // _CH // _NW
    r_base = wid * n * _CH

    ins = [in_a, in_b]
    outs = [out_a, out_b]
    sem_in = [si_a, si_b]
    sem_out = [so_a, so_b]

    def start_in(i, b):
        r0 = r_base + i * _CH
        pltpu.async_copy(x_hbm.at[pl.ds(r0, _CH)], ins[b], sem_in[b])

    def start_out(i, b):
        r0 = r_base + i * _CH
        pltpu.async_copy(outs[b], out_hbm.at[pl.ds(r0, _CH)], sem_out[b])

    def wait_in(b):
        pltpu.make_async_copy(
            x_hbm.at[pl.ds(0, _CH)], ins[b], sem_in[b]
        ).wait()

    def wait_out(b):
        pltpu.make_async_copy(
            outs[b], out_hbm.at[pl.ds(0, _CH)], sem_out[b]
        ).wait()

    start_in(0, 0)
    start_in(1, 1)
    wait_in(0)
    _permute_chunk(ins[0], outs[0])
    start_out(0, 0)
    start_in(2, 0)
    wait_in(1)
    _permute_chunk(ins[1], outs[1])
    start_out(1, 1)
    start_in(3, 1)

    def pair_body(g, _):
        i0 = 2 * g
        wait_in(0)
        wait_out(0)
        _permute_chunk(ins[0], outs[0])
        start_out(i0, 0)
        start_in(i0 + 2, 0)
        wait_in(1)
        wait_out(1)
        _permute_chunk(ins[1], outs[1])
        start_out(i0 + 1, 1)
        start_in(i0 + 3, 1)
        return 0

    lax.fori_loop(1, n // 2 - 1, pair_body, 0)

    wait_in(0)
    wait_out(0)
    _permute_chunk(ins[0], outs[0])
    start_out(n - 2, 0)
    wait_in(1)
    wait_out(1)
    _permute_chunk(ins[1], outs[1])
    start_out(n - 1, 1)
    wait_out(0)
    wait_out(1)


def kernel(inputs):
    b, t, f = inputs.shape
    xt = jnp.transpose(inputs, (1, 0, 2)).reshape(t * b, f)
    mesh = plsc.VectorSubcoreMesh(core_axis_name="c", subcore_axis_name="s")
    k = functools.partial(
        pl.kernel,
        out_type=jax.ShapeDtypeStruct((t * b, f), jnp.float32),
        mesh=mesh,
        scratch_types=[
            pltpu.VMEM((_CH, f), jnp.float32),
            pltpu.VMEM((_CH, f), jnp.float32),
            pltpu.VMEM((_CH, f), jnp.float32),
            pltpu.VMEM((_CH, f), jnp.float32),
            pltpu.SemaphoreType.DMA,
            pltpu.SemaphoreType.DMA,
            pltpu.SemaphoreType.DMA,
            pltpu.SemaphoreType.DMA,
        ],
        compiler_params=pltpu.CompilerParams(
            needs_layout_passes=False, use_tc_tiling_on_sc=True
        ),
    )(_sc_body)
    return jnp.transpose(k(xt).reshape(t, b, f), (1, 0, 2))


# speedup vs baseline: 1.0108x; 1.0012x over previous
import functools

import jax
import jax.numpy as jnp
from jax import lax
from jax.experimental import pallas as pl
from jax.experimental.pallas import tpu as pltpu
from jax.experimental.pallas import tpu_sc as plsc

_NUM_CCSK = 8
_SEQ = 128
_F = _NUM_CCSK * _SEQ
_NC = 2
_NS = 16
_NW = _NC * _NS
_CH = 16


def _permute_chunk(in_v, out_v):
    pattern = lax.iota(jnp.int32, 16) << 3

    def q_body(q, _):
        r = q >> 3
        jh = q & 7
        rvec = jnp.broadcast_to(r, (16,))
        vs = [
            in_v[r, pl.ds(k * _SEQ + 16 * jh, 16)]
            for k in range(_NUM_CCSK)
        ]
        for k in range(_NUM_CCSK):
            plsc.store_scatter(
                out_v, [rvec, pattern + (jh * _SEQ + k)], vs[k]
            )
        return 0

    lax.fori_loop(0, _CH * _NUM_CCSK, q_body, 0, unroll=2)


def _sc_body(x_hbm, out_hbm, in_a, in_b, out_a, out_b, si_a, si_b, so_a, so_b):
    wid = lax.axis_index("s") * _NC + lax.axis_index("c")
    rows = x_hbm.shape[0]
    n = rows // _CH // _NW
    r_base = wid * n * _CH

    ins = [in_a, in_b]
    outs = [out_a, out_b]
    sem_in = [si_a, si_b]
    sem_out = [so_a, so_b]

    def start_in(i, b):
        r0 = r_base + i * _CH
        pltpu.async_copy(x_hbm.at[pl.ds(r0, _CH)], ins[b], sem_in[b])

    def start_out(i, b):
        r0 = r_base + i * _CH
        pltpu.async_copy(outs[b], out_hbm.at[pl.ds(r0, _CH)], sem_out[b])

    def wait_in(b):
        pltpu.make_async_copy(
            x_hbm.at[pl.ds(0, _CH)], ins[b], sem_in[b]
        ).wait()

    def wait_out(b):
        pltpu.make_async_copy(
            outs[b], out_hbm.at[pl.ds(0, _CH)], sem_out[b]
        ).wait()

    start_in(0, 0)
    start_in(1, 1)
    wait_in(0)
    _permute_chunk(ins[0], outs[0])
    start_out(0, 0)
    start_in(2, 0)
    wait_in(1)
    _permute_chunk(ins[1], outs[1])
    start_out(1, 1)
    start_in(3, 1)

    def pair_body(g, _):
        i0 = 2 * g
        wait_in(0)
        wait_out(0)
        _permute_chunk(ins[0], outs[0])
        start_out(i0, 0)
        start_in(i0 + 2, 0)
        wait_in(1)
        wait_out(1)
        _permute_chunk(ins[1], outs[1])
        start_out(i0 + 1, 1)
        start_in(i0 + 3, 1)
        return 0

    lax.fori_loop(1, n // 2 - 1, pair_body, 0)

    wait_in(0)
    wait_out(0)
    _permute_chunk(ins[0], outs[0])
    start_out(n - 2, 0)
    wait_in(1)
    wait_out(1)
    _permute_chunk(ins[1], outs[1])
    start_out(n - 1, 1)
    wait_out(0)
    wait_out(1)


def kernel(inputs):
    b, t, f = inputs.shape
    xt = jnp.transpose(inputs, (1, 0, 2)).reshape(t * b, f)
    mesh = plsc.VectorSubcoreMesh(core_axis_name="c", subcore_axis_name="s")
    k = functools.partial(
        pl.kernel,
        out_type=jax.ShapeDtypeStruct((t * b, f), jnp.float32),
        mesh=mesh,
        scratch_types=[
            pltpu.VMEM((_CH, f), jnp.float32),
            pltpu.VMEM((_CH, f), jnp.float32),
            pltpu.VMEM((_CH, f), jnp.float32),
            pltpu.VMEM((_CH, f), jnp.float32),
            pltpu.SemaphoreType.DMA,
            pltpu.SemaphoreType.DMA,
            pltpu.SemaphoreType.DMA,
            pltpu.SemaphoreType.DMA,
        ],
        compiler_params=pltpu.CompilerParams(
            needs_layout_passes=False,
            use_tc_tiling_on_sc=True,
            skip_device_barrier=True,
        ),
    )(_sc_body)
    return jnp.transpose(k(xt).reshape(t, b, f), (1, 0, 2))
